# baseline (speedup 1.0000x reference)
import functools

import jax
import jax.numpy as jnp
from jax import lax
from jax.experimental import pallas as pl
from jax.experimental.pallas import tpu as pltpu, tpu_sc as plsc

V = 1000000
D = 64
NB = 16384
NS = 50
B_TOTAL = NB * NS
NW = 32
B_PER_W = B_TOTAL // NW

_mesh = plsc.VectorSubcoreMesh(core_axis_name="c", subcore_axis_name="s")

TBLK = 4608


def _tt_body(wt_ref, o_ref, s_ref):
    s_ref[...] = wt_ref[...].T
    o_ref[:, 0:64] = s_ref[0::2, :]
    o_ref[:, 64:128] = s_ref[1::2, :]


_t_kernel = pl.pallas_call(
    _tt_body,
    grid=((V + TBLK - 1) // TBLK,),
    in_specs=[pl.BlockSpec((D, TBLK), lambda j: (0, j))],
    out_specs=pl.BlockSpec((TBLK // 2, 128), lambda j: (j, 0)),
    out_shape=jax.ShapeDtypeStruct((V // 2, 128), jnp.float32),
    scratch_shapes=[pltpu.VMEM((TBLK, 64), jnp.float32)],
    compiler_params=pltpu.CompilerParams(
        dimension_semantics=("parallel",)),
)


N_BLK2 = 4 * NS


@functools.partial(
    pl.kernel,
    out_type=jax.ShapeDtypeStruct((NS, 8, 128, 8, 128), jnp.float32),
    mesh=_mesh,
    compiler_params=pltpu.CompilerParams(use_tc_tiling_on_sc=False,
                                         needs_layout_passes=False),
    scratch_types=[
        pltpu.VMEM((B_PER_W,), jnp.int32),
        [pltpu.VMEM((128,), jnp.int32)] * 2,
        [pltpu.VMEM((128, D), jnp.float32)] * 2,
        [pltpu.VMEM((8, 8, 128), jnp.float32)] * 2,
        pltpu.VMEM((D,), jnp.float32),
        [pltpu.SemaphoreType.DMA] * 2,
        [pltpu.SemaphoreType.DMA] * 2,
    ],
)
def _g_kernel(x_hbm, w_hbm, b_hbm, out_hbm, idx_v, gidx, rowbuf, oblk,
              bias_v, sg, ss):
    wid = lax.axis_index("s") * 2 + lax.axis_index("c")
    base = wid * B_PER_W

    pltpu.sync_copy(b_hbm, bias_v)
    pltpu.sync_copy(x_hbm.at[pl.ds(base, B_PER_W)], idx_v)

    p50 = lax.iota(jnp.int32, 16) * 50
    rowvecs = [lax.iota(jnp.int32, 16) + 16 * g for g in range(8)]

    def build_gidx(t, b):
        s = t % 50
        cbl = t // 50
        for g in range(8):
            pos = p50 + lax.broadcast(cbl * 6400 + g * 800 + s, (16,))
            vals = plsc.load_gather(idx_v, [pos])
            gidx[b][pl.ds(g * 16, 16)] = vals

    def fire_gather(b):
        pltpu.async_copy(w_hbm.at[gidx[b]], rowbuf[b], sg[b])

    def wait_gather(b):
        pltpu.make_async_copy(w_hbm.at[pl.ds(0, 128)], rowbuf[b],
                              sg[b]).wait()

    def fire_store(t, b):
        s = t % 50
        cb = wid * 4 + t // 50
        pltpu.async_copy(oblk[b], out_hbm.at[s, :, cb], ss[b])

    def wait_store(b):
        pltpu.make_async_copy(oblk[b], out_hbm.at[0, :, 0], ss[b]).wait()

    def transpose_bias(b):
        @plsc.parallel_loop(0, D, unroll=8)
        def d_body(d):
            dsplat = lax.broadcast(d, (16,))
            bsplat = plsc.load_gather(bias_v, [dsplat])
            for g in range(8):
                rv = plsc.load_gather(rowbuf[b], [rowvecs[g], dsplat])
                oblk[b][d // 8, d % 8, pl.ds(g * 16, 16)] = rv + bsplat

    def visit(t, b, first):
        wait_gather(b)
        build_gidx(jnp.minimum(t + 1, N_BLK2 - 1), 1 - b)
        fire_gather(1 - b)
        if not first:
            wait_store(b)
        transpose_bias(b)
        fire_store(t, b)

    build_gidx(0, 0)
    fire_gather(0)
    visit(0, 0, True)
    visit(1, 1, True)

    def outer(p, carry):
        def visit_l(t, b):
            wait_gather(b)
            build_gidx(jnp.minimum(t + 1, N_BLK2 - 1), 1 - b)
            fire_gather(1 - b)
            wait_store(b)
            transpose_bias(b)
            fire_store(t, b)

        visit_l(2 * p, 0)
        visit_l(2 * p + 1, 1)
        return carry

    lax.fori_loop(1, N_BLK2 // 2, outer, 0)

    wait_gather(0)
    wait_store(0)
    wait_store(1)


def kernel(x, weight, bias):
    w2 = _t_kernel(weight.T)
    w_lin = w2.reshape(V, D)
    out5 = _g_kernel(x.reshape(-1), w_lin, bias)
    return out5.transpose(2, 4, 0, 1, 3).reshape(NB, NS, D)


# speedup vs baseline: 1.0535x; 1.0535x over previous
import functools

import jax
import jax.numpy as jnp
from jax import lax
from jax.experimental import pallas as pl
from jax.experimental.pallas import tpu as pltpu, tpu_sc as plsc

V = 1000000
D = 64
NB = 16384
NS = 50
B_TOTAL = NB * NS
NW = 32
B_PER_W = B_TOTAL // NW

_mesh = plsc.VectorSubcoreMesh(core_axis_name="c", subcore_axis_name="s")

TBLK = 9216


def _tt_body(wt_ref, o_ref, s_ref):
    s_ref[...] = wt_ref[...].T
    o_ref[:, 0:64] = s_ref[0::2, :]
    o_ref[:, 64:128] = s_ref[1::2, :]


_t_kernel = pl.pallas_call(
    _tt_body,
    grid=((V + TBLK - 1) // TBLK,),
    in_specs=[pl.BlockSpec((D, TBLK), lambda j: (0, j))],
    out_specs=pl.BlockSpec((TBLK // 2, 128), lambda j: (j, 0)),
    out_shape=jax.ShapeDtypeStruct((V // 2, 128), jnp.float32),
    scratch_shapes=[pltpu.VMEM((TBLK, 64), jnp.float32)],
    compiler_params=pltpu.CompilerParams(
        dimension_semantics=("parallel",)),
)


N_BLK2 = 4 * NS


@functools.partial(
    pl.kernel,
    out_type=jax.ShapeDtypeStruct((NS, 8, 128, 8, 128), jnp.float32),
    mesh=_mesh,
    compiler_params=pltpu.CompilerParams(use_tc_tiling_on_sc=False,
                                         needs_layout_passes=False),
    scratch_types=[
        pltpu.VMEM((B_PER_W,), jnp.int32),
        [pltpu.VMEM((128,), jnp.int32)] * 2,
        [pltpu.VMEM((128, D), jnp.float32)] * 2,
        [pltpu.VMEM((8, 8, 128), jnp.float32)] * 2,
        pltpu.VMEM((D,), jnp.float32),
        [pltpu.SemaphoreType.DMA] * 2,
        [pltpu.SemaphoreType.DMA] * 2,
    ],
)
def _g_kernel(x_hbm, w_hbm, b_hbm, out_hbm, idx_v, gidx, rowbuf, oblk,
              bias_v, sg, ss):
    wid = lax.axis_index("s") * 2 + lax.axis_index("c")
    base = wid * B_PER_W

    pltpu.sync_copy(b_hbm, bias_v)
    pltpu.sync_copy(x_hbm.at[pl.ds(base, B_PER_W)], idx_v)

    p50 = lax.iota(jnp.int32, 16) * 50
    rowvecs = [lax.iota(jnp.int32, 16) + 16 * g for g in range(8)]

    def build_gidx(t, b):
        s = t % 50
        cbl = t // 50
        for g in range(8):
            pos = p50 + lax.broadcast(cbl * 6400 + g * 800 + s, (16,))
            vals = plsc.load_gather(idx_v, [pos])
            gidx[b][pl.ds(g * 16, 16)] = vals

    def fire_gather(b):
        pltpu.async_copy(w_hbm.at[gidx[b]], rowbuf[b], sg[b])

    def wait_gather(b):
        pltpu.make_async_copy(w_hbm.at[pl.ds(0, 128)], rowbuf[b],
                              sg[b]).wait()

    def fire_store(t, b):
        s = t % 50
        cb = wid * 4 + t // 50
        pltpu.async_copy(oblk[b], out_hbm.at[s, :, cb], ss[b])

    def wait_store(b):
        pltpu.make_async_copy(oblk[b], out_hbm.at[0, :, 0], ss[b]).wait()

    def transpose_bias(b):
        @plsc.parallel_loop(0, D, unroll=8)
        def d_body(d):
            dsplat = lax.broadcast(d, (16,))
            bsplat = plsc.load_gather(bias_v, [dsplat])
            for g in range(8):
                rv = plsc.load_gather(rowbuf[b], [rowvecs[g], dsplat])
                oblk[b][d // 8, d % 8, pl.ds(g * 16, 16)] = rv + bsplat

    def visit(t, b, first):
        wait_gather(b)
        build_gidx(jnp.minimum(t + 1, N_BLK2 - 1), 1 - b)
        fire_gather(1 - b)
        if not first:
            wait_store(b)
        transpose_bias(b)
        fire_store(t, b)

    build_gidx(0, 0)
    fire_gather(0)
    visit(0, 0, True)
    visit(1, 1, True)

    def outer(p, carry):
        def visit_l(t, b):
            wait_gather(b)
            build_gidx(jnp.minimum(t + 1, N_BLK2 - 1), 1 - b)
            fire_gather(1 - b)
            wait_store(b)
            transpose_bias(b)
            fire_store(t, b)

        visit_l(2 * p, 0)
        visit_l(2 * p + 1, 1)
        return carry

    lax.fori_loop(1, N_BLK2 // 2, outer, 0)

    wait_gather(0)
    wait_store(0)
    wait_store(1)


def kernel(x, weight, bias):
    w2 = _t_kernel(weight.T)
    w_lin = w2.reshape(V, D)
    out5 = _g_kernel(x.reshape(-1), w_lin, bias)
    return out5.transpose(2, 4, 0, 1, 3).reshape(NB, NS, D)


# speedup vs baseline: 1.0625x; 1.0085x over previous
import functools

import jax
import jax.numpy as jnp
from jax import lax
from jax.experimental import pallas as pl
from jax.experimental.pallas import tpu as pltpu, tpu_sc as plsc

V = 1000000
D = 64
NB = 16384
NS = 50
B_TOTAL = NB * NS
NW = 32
B_PER_W = B_TOTAL // NW

_mesh = plsc.VectorSubcoreMesh(core_axis_name="c", subcore_axis_name="s")

TBLK = 18432


def _tt_body(wt_ref, o_ref, s_ref):
    s_ref[...] = wt_ref[...].T
    o_ref[:, 0:64] = s_ref[0::2, :]
    o_ref[:, 64:128] = s_ref[1::2, :]


_t_kernel = pl.pallas_call(
    _tt_body,
    grid=((V + TBLK - 1) // TBLK,),
    in_specs=[pl.BlockSpec((D, TBLK), lambda j: (0, j))],
    out_specs=pl.BlockSpec((TBLK // 2, 128), lambda j: (j, 0)),
    out_shape=jax.ShapeDtypeStruct((V // 2, 128), jnp.float32),
    scratch_shapes=[pltpu.VMEM((TBLK, 64), jnp.float32)],
    compiler_params=pltpu.CompilerParams(
        dimension_semantics=("parallel",)),
)


N_BLK2 = 4 * NS


@functools.partial(
    pl.kernel,
    out_type=jax.ShapeDtypeStruct((NS, 8, 128, 8, 128), jnp.float32),
    mesh=_mesh,
    compiler_params=pltpu.CompilerParams(use_tc_tiling_on_sc=False,
                                         needs_layout_passes=False),
    scratch_types=[
        pltpu.VMEM((B_PER_W,), jnp.int32),
        [pltpu.VMEM((128,), jnp.int32)] * 2,
        [pltpu.VMEM((128, D), jnp.float32)] * 2,
        [pltpu.VMEM((8, 8, 128), jnp.float32)] * 2,
        pltpu.VMEM((D,), jnp.float32),
        [pltpu.SemaphoreType.DMA] * 2,
        [pltpu.SemaphoreType.DMA] * 2,
    ],
)
def _g_kernel(x_hbm, w_hbm, b_hbm, out_hbm, idx_v, gidx, rowbuf, oblk,
              bias_v, sg, ss):
    wid = lax.axis_index("s") * 2 + lax.axis_index("c")
    base = wid * B_PER_W

    pltpu.sync_copy(b_hbm, bias_v)
    pltpu.sync_copy(x_hbm.at[pl.ds(base, B_PER_W)], idx_v)

    p50 = lax.iota(jnp.int32, 16) * 50
    rowvecs = [lax.iota(jnp.int32, 16) + 16 * g for g in range(8)]

    def build_gidx(t, b):
        s = t % 50
        cbl = t // 50
        for g in range(8):
            pos = p50 + lax.broadcast(cbl * 6400 + g * 800 + s, (16,))
            vals = plsc.load_gather(idx_v, [pos])
            gidx[b][pl.ds(g * 16, 16)] = vals

    def fire_gather(b):
        pltpu.async_copy(w_hbm.at[gidx[b]], rowbuf[b], sg[b])

    def wait_gather(b):
        pltpu.make_async_copy(w_hbm.at[pl.ds(0, 128)], rowbuf[b],
                              sg[b]).wait()

    def fire_store(t, b):
        s = t % 50
        cb = wid * 4 + t // 50
        pltpu.async_copy(oblk[b], out_hbm.at[s, :, cb], ss[b])

    def wait_store(b):
        pltpu.make_async_copy(oblk[b], out_hbm.at[0, :, 0], ss[b]).wait()

    def transpose_bias(b):
        @plsc.parallel_loop(0, D, unroll=8)
        def d_body(d):
            dsplat = lax.broadcast(d, (16,))
            bsplat = plsc.load_gather(bias_v, [dsplat])
            for g in range(8):
                rv = plsc.load_gather(rowbuf[b], [rowvecs[g], dsplat])
                oblk[b][d // 8, d % 8, pl.ds(g * 16, 16)] = rv + bsplat

    def visit(t, b, first):
        wait_gather(b)
        build_gidx(jnp.minimum(t + 1, N_BLK2 - 1), 1 - b)
        fire_gather(1 - b)
        if not first:
            wait_store(b)
        transpose_bias(b)
        fire_store(t, b)

    build_gidx(0, 0)
    fire_gather(0)
    visit(0, 0, True)
    visit(1, 1, True)

    def outer(p, carry):
        def visit_l(t, b):
            wait_gather(b)
            build_gidx(jnp.minimum(t + 1, N_BLK2 - 1), 1 - b)
            fire_gather(1 - b)
            wait_store(b)
            transpose_bias(b)
            fire_store(t, b)

        visit_l(2 * p, 0)
        visit_l(2 * p + 1, 1)
        return carry

    lax.fori_loop(1, N_BLK2 // 2, outer, 0)

    wait_gather(0)
    wait_store(0)
    wait_store(1)


def kernel(x, weight, bias):
    w2 = _t_kernel(weight.T)
    w_lin = w2.reshape(V, D)
    out5 = _g_kernel(x.reshape(-1), w_lin, bias)
    return out5.transpose(2, 4, 0, 1, 3).reshape(NB, NS, D)
